# Initial kernel scaffold; baseline (speedup 1.0000x reference)
import jax
import jax.numpy as jnp
from jax.experimental import pallas as pl


def kernel(x, pitch_emb, velocity_emb, onset_W, onset_b, dur_W, dur_b, proj_W, proj_b):
    raise NotImplementedError("write your pallas kernel here")


# speedup vs baseline: 4.6269x; 4.6269x over previous
import functools

import jax
import jax.numpy as jnp
from jax import lax
from jax.experimental import pallas as pl
from jax.experimental.pallas import tpu as pltpu
from jax.experimental.pallas import tpu_sc as plsc

NC, NS, LANES = 2, 16, 16
NW = NC * NS
D = 128
E = 64
TBLK = 128


def _prep_body(pitch_ref, vel_ref, ow_ref, ob_ref, dw_ref, db_ref, pw_ref,
               pb_ref, p2_ref, v2_ref, aux_ref):
    w1 = pw_ref[0:E, :]
    w2 = pw_ref[E:2 * E, :]
    w3 = pw_ref[2 * E:3 * E, :]
    w4 = pw_ref[3 * E:4 * E, :]
    const = (pb_ref[...]
             + jnp.dot(ob_ref[...], w2, preferred_element_type=jnp.float32)
             + jnp.dot(db_ref[...], w3, preferred_element_type=jnp.float32))
    p2_ref[...] = jnp.dot(pitch_ref[...], w1,
                          preferred_element_type=jnp.float32) + const
    v2_ref[...] = jnp.dot(vel_ref[...], w4,
                          preferred_element_type=jnp.float32)
    owv = jnp.dot(ow_ref[...], w2, preferred_element_type=jnp.float32)
    dwv = jnp.dot(dw_ref[...], w3, preferred_element_type=jnp.float32)
    aux_ref[...] = jnp.concatenate([owv, dwv], axis=0)


def _prep(pitch_emb, velocity_emb, onset_W, onset_b, dur_W, dur_b, proj_W,
          proj_b):
    pv = pitch_emb.shape[0]
    vv = velocity_emb.shape[0]
    return pl.pallas_call(
        _prep_body,
        out_shape=(
            jax.ShapeDtypeStruct((pv, D), jnp.float32),
            jax.ShapeDtypeStruct((vv, D), jnp.float32),
            jax.ShapeDtypeStruct((2, D), jnp.float32),
        ),
    )(pitch_emb, velocity_emb, onset_W, onset_b.reshape(1, E), dur_W,
      dur_b.reshape(1, E), proj_W, proj_b.reshape(1, D))


def _bcast_lane(vec, lane):
    idx = jnp.full((LANES,), lane, jnp.int32)
    return lax.gather(
        vec, idx[:, None],
        lax.GatherDimensionNumbers(offset_dims=(), collapsed_slice_dims=(0,),
                                   start_index_map=(0,)),
        (1,), mode=lax.GatherScatterMode.PROMISE_IN_BOUNDS)


def _make_sc_kernel(n_tokens):
    tw = n_tokens // NW
    nblk = tw // TBLK
    mesh = plsc.VectorSubcoreMesh(core_axis_name="c", subcore_axis_name="s",
                                  num_cores=NC, num_subcores=NS)

    @functools.partial(
        pl.kernel,
        out_type=jax.ShapeDtypeStruct((n_tokens, D), jnp.float32),
        mesh=mesh,
        scratch_types=[
            pltpu.VMEM((TBLK,), jnp.int32),
            pltpu.VMEM((TBLK,), jnp.int32),
            pltpu.VMEM((TBLK,), jnp.float32),
            pltpu.VMEM((TBLK,), jnp.float32),
            pltpu.VMEM((TBLK, D), jnp.float32),
            pltpu.VMEM((TBLK, D), jnp.float32),
            pltpu.VMEM((D,), jnp.float32),
            pltpu.VMEM((D,), jnp.float32),
            pltpu.SemaphoreType.DMA,
            pltpu.SemaphoreType.DMA,
        ],
    )
    def sc_kernel(p2_hbm, v2_hbm, owdw_hbm, pidx_hbm, vidx_hbm, onset_hbm,
                  dur_hbm, out_hbm, pidx_v, vidx_v, onset_v, dur_v, bufp,
                  bufv, ow_v, dw_v, sem1, sem2):
        wid = lax.axis_index("s") * NC + lax.axis_index("c")
        pltpu.sync_copy(owdw_hbm.at[0], ow_v)
        pltpu.sync_copy(owdw_hbm.at[1], dw_v)
        ow_regs = [ow_v[pl.ds(LANES * j, LANES)] for j in range(D // LANES)]
        dw_regs = [dw_v[pl.ds(LANES * j, LANES)] for j in range(D // LANES)]

        def block(g, carry):
            base = wid * tw + g * TBLK
            pltpu.sync_copy(pidx_hbm.at[pl.ds(base, TBLK)], pidx_v)
            pltpu.sync_copy(vidx_hbm.at[pl.ds(base, TBLK)], vidx_v)
            pltpu.sync_copy(onset_hbm.at[pl.ds(base, TBLK)], onset_v)
            pltpu.sync_copy(dur_hbm.at[pl.ds(base, TBLK)], dur_v)
            cp1 = pltpu.async_copy(p2_hbm.at[pidx_v], bufp, sem1)
            cp2 = pltpu.async_copy(v2_hbm.at[vidx_v], bufv, sem2)
            cp1.wait()
            cp2.wait()

            def group(t16, tc):
                tb = t16 * LANES
                ov = onset_v[pl.ds(tb, LANES)]
                dv = dur_v[pl.ds(tb, LANES)]
                for lane in range(LANES):
                    t = tb + lane
                    bo = _bcast_lane(ov, lane)
                    bd = _bcast_lane(dv, lane)
                    for j in range(D // LANES):
                        js = pl.ds(LANES * j, LANES)
                        bufp[t, js] = (bufp[t, js] + bufv[t, js]
                                       + bo * ow_regs[j] + bd * dw_regs[j])
                return tc

            lax.fori_loop(0, TBLK // LANES, group, 0, unroll=False)
            pltpu.sync_copy(bufp, out_hbm.at[pl.ds(base, TBLK)])
            return carry

        lax.fori_loop(0, nblk, block, 0, unroll=False)

    return sc_kernel


def kernel(x, pitch_emb, velocity_emb, onset_W, onset_b, dur_W, dur_b,
           proj_W, proj_b):
    b, l, _ = x.shape
    n = b * l
    p2, v2, aux = _prep(pitch_emb, velocity_emb, onset_W, onset_b, dur_W,
                        dur_b, proj_W, proj_b)
    xf = x.reshape(n, 4)
    pidx = xf[:, 0].astype(jnp.int32)
    vidx = xf[:, 3].astype(jnp.int32)
    onset = xf[:, 1]
    dur = xf[:, 2]
    out = _make_sc_kernel(n)(p2, v2, aux, pidx, vidx, onset, dur)
    return out.reshape(b, l, D)


# speedup vs baseline: 4.9195x; 1.0632x over previous
import functools

import jax
import jax.numpy as jnp
from jax import lax
from jax.experimental import pallas as pl
from jax.experimental.pallas import tpu as pltpu
from jax.experimental.pallas import tpu_sc as plsc

NC, NS, LANES = 2, 16, 16
NW = NC * NS
D = 128
E = 64
TBLK = 128
NJ = D // LANES


def _prep_body(pitch_ref, vel_ref, ow_ref, ob_ref, dw_ref, db_ref, pw_ref,
               pb_ref, p2_ref, v2_ref, aux_ref):
    w1 = pw_ref[0:E, :]
    w2 = pw_ref[E:2 * E, :]
    w3 = pw_ref[2 * E:3 * E, :]
    w4 = pw_ref[3 * E:4 * E, :]
    const = (pb_ref[...]
             + jnp.dot(ob_ref[...], w2, preferred_element_type=jnp.float32)
             + jnp.dot(db_ref[...], w3, preferred_element_type=jnp.float32))
    p2_ref[...] = jnp.dot(pitch_ref[...], w1,
                          preferred_element_type=jnp.float32) + const
    v2_ref[...] = jnp.dot(vel_ref[...], w4,
                          preferred_element_type=jnp.float32)
    owv = jnp.dot(ow_ref[...], w2, preferred_element_type=jnp.float32)
    dwv = jnp.dot(dw_ref[...], w3, preferred_element_type=jnp.float32)
    aux_ref[...] = jnp.concatenate([owv, dwv], axis=0)


def _prep(pitch_emb, velocity_emb, onset_W, onset_b, dur_W, dur_b, proj_W,
          proj_b):
    pv = pitch_emb.shape[0]
    vv = velocity_emb.shape[0]
    return pl.pallas_call(
        _prep_body,
        out_shape=(
            jax.ShapeDtypeStruct((pv, D), jnp.float32),
            jax.ShapeDtypeStruct((vv, D), jnp.float32),
            jax.ShapeDtypeStruct((2, D), jnp.float32),
        ),
    )(pitch_emb, velocity_emb, onset_W, onset_b.reshape(1, E), dur_W,
      dur_b.reshape(1, E), proj_W, proj_b.reshape(1, D))


_GDN = lax.GatherDimensionNumbers(offset_dims=(), collapsed_slice_dims=(0,),
                                  start_index_map=(0,))


def _dyn_gather(vec, idx):
    return lax.gather(vec, idx[:, None], _GDN, (1,),
                      mode=lax.GatherScatterMode.PROMISE_IN_BOUNDS)


def _splat(vec, lane):
    return _dyn_gather(vec, jnp.full((LANES,), lane, jnp.int32))


def _make_sc_kernel(nblk_total):
    assert nblk_total % NW == 0
    nblk = nblk_total // NW
    assert nblk % 2 == 0
    n_tokens = nblk_total * TBLK
    mesh = plsc.VectorSubcoreMesh(core_axis_name="c", subcore_axis_name="s",
                                  num_cores=NC, num_subcores=NS)

    @functools.partial(
        pl.kernel,
        out_type=jax.ShapeDtypeStruct((n_tokens, D), jnp.float32),
        mesh=mesh,
        scratch_types=[
            pltpu.VMEM((2, 2, TBLK), jnp.int32),
            pltpu.VMEM((2, 2, TBLK), jnp.float32),
            pltpu.VMEM((2, TBLK, D), jnp.float32),
            pltpu.VMEM((2, TBLK, D), jnp.float32),
            pltpu.VMEM((D,), jnp.float32),
            pltpu.VMEM((D,), jnp.float32),
            pltpu.SemaphoreType.DMA((2,)),
            pltpu.SemaphoreType.DMA((2,)),
            pltpu.SemaphoreType.DMA((2,)),
            pltpu.SemaphoreType.DMA((2,)),
            pltpu.SemaphoreType.DMA((2,)),
        ],
    )
    def sc_kernel(p2_hbm, v2_hbm, owdw_hbm, idx_hbm, sc_hbm, out_hbm,
                  idxb, scb, bufp, bufv, ow_v, dw_v,
                  sem_i, sem_s, sem_gp, sem_gv, sem_o):
        cid = lax.axis_index("c")
        sid = lax.axis_index("s")
        wid = sid * NC + cid

        pltpu.sync_copy(owdw_hbm.at[0], ow_v)
        pltpu.sync_copy(owdw_hbm.at[1], dw_v)
        ow_regs = [ow_v[pl.ds(LANES * j, LANES)] for j in range(NJ)]
        dw_regs = [dw_v[pl.ds(LANES * j, LANES)] for j in range(NJ)]
        blk0 = wid * nblk

        def i_copy(g, s):
            return pltpu.make_async_copy(idx_hbm.at[blk0 + g], idxb.at[s],
                                         sem_i.at[s])

        def s_copy(g, s):
            return pltpu.make_async_copy(sc_hbm.at[blk0 + g], scb.at[s],
                                         sem_s.at[s])

        def gp_copy(s):
            return pltpu.make_async_copy(p2_hbm.at[idxb.at[s].at[0]],
                                         bufp.at[s], sem_gp.at[s])

        def gv_copy(s):
            return pltpu.make_async_copy(v2_hbm.at[idxb.at[s].at[1]],
                                         bufv.at[s], sem_gv.at[s])

        def o_copy(g, s):
            return pltpu.make_async_copy(
                bufp.at[s],
                out_hbm.at[pl.ds((blk0 + g) * TBLK, TBLK)],
                sem_o.at[s])

        def compute(s):
            bp = bufp.at[s]
            bv = bufv.at[s]

            def group(t16, cc):
                tb = t16 * LANES
                ov = scb[s, 0, pl.ds(tb, LANES)]
                dv = scb[s, 1, pl.ds(tb, LANES)]
                for lane in range(LANES):
                    t = tb + lane
                    bo = _splat(ov, lane)
                    bd = _splat(dv, lane)
                    for j in range(NJ):
                        js = pl.ds(LANES * j, LANES)
                        bp[t, js] = (bp[t, js] + bv[t, js]
                                     + bo * ow_regs[j] + bd * dw_regs[j])
                return cc

            lax.fori_loop(0, TBLK // LANES, group, 0)

        i_copy(0, 0).start()
        s_copy(0, 0).start()
        i_copy(1, 1).start()
        s_copy(1, 1).start()
        i_copy(0, 0).wait()
        gp_copy(0).start()
        gv_copy(0).start()

        def block(gg, carry):
            for s in range(2):
                g = 2 * gg + s
                s1 = 1 - s

                @pl.when(g + 1 < nblk)
                def _next():
                    i_copy(g + 1, s1).wait()

                    @pl.when(g >= 1)
                    def _drain_out():
                        o_copy(g - 1, s1).wait()

                    gp_copy(s1).start()
                    gv_copy(s1).start()

                gp_copy(s).wait()
                gv_copy(s).wait()
                s_copy(g, s).wait()
                compute(s)

                @pl.when(g + 2 < nblk)
                def _fetch_next():
                    i_copy(g + 2, s).start()
                    s_copy(g + 2, s).start()

                o_copy(g, s).start()
            return carry

        lax.fori_loop(0, nblk // 2, block, 0)
        o_copy(nblk - 2, 0).wait()
        o_copy(nblk - 1, 1).wait()

    return sc_kernel


def kernel(x, pitch_emb, velocity_emb, onset_W, onset_b, dur_W, dur_b,
           proj_W, proj_b):
    b, l, _ = x.shape
    n = b * l
    nblk_total = n // TBLK
    p2, v2, aux = _prep(pitch_emb, velocity_emb, onset_W, onset_b, dur_W,
                        dur_b, proj_W, proj_b)
    xf = x.reshape(nblk_total, TBLK, 4)
    idx2 = jnp.stack([xf[:, :, 0], xf[:, :, 3]], axis=1).astype(jnp.int32)
    sc2 = jnp.stack([xf[:, :, 1], xf[:, :, 2]], axis=1)
    out = _make_sc_kernel(nblk_total)(p2, v2, aux, idx2, sc2)
    return out.reshape(b, l, D)


# speedup vs baseline: 11.6991x; 2.3781x over previous
import functools

import jax
import jax.numpy as jnp
from jax import lax
from jax.experimental import pallas as pl
from jax.experimental.pallas import tpu as pltpu
from jax.experimental.pallas import tpu_sc as plsc

NC, NS, LANES = 2, 16, 16
NW = NC * NS
D = 128
E = 64
TBLK = 128
NJ = D // LANES


def _prep_body(pitch_ref, vel_ref, ow_ref, ob_ref, dw_ref, db_ref, pw_ref,
               pb_ref, p2_ref, v2_ref, aux_ref):
    w1 = pw_ref[0:E, :]
    w2 = pw_ref[E:2 * E, :]
    w3 = pw_ref[2 * E:3 * E, :]
    w4 = pw_ref[3 * E:4 * E, :]
    const = (pb_ref[...]
             + jnp.dot(ob_ref[...], w2, preferred_element_type=jnp.float32)
             + jnp.dot(db_ref[...], w3, preferred_element_type=jnp.float32))
    p2_ref[...] = jnp.dot(pitch_ref[...], w1,
                          preferred_element_type=jnp.float32) + const
    v2_ref[...] = jnp.dot(vel_ref[...], w4,
                          preferred_element_type=jnp.float32)
    owv = jnp.dot(ow_ref[...], w2, preferred_element_type=jnp.float32)
    dwv = jnp.dot(dw_ref[...], w3, preferred_element_type=jnp.float32)
    aux_ref[...] = jnp.concatenate([owv, dwv], axis=0)


def _prep(pitch_emb, velocity_emb, onset_W, onset_b, dur_W, dur_b, proj_W,
          proj_b):
    pv = pitch_emb.shape[0]
    vv = velocity_emb.shape[0]
    return pl.pallas_call(
        _prep_body,
        out_shape=(
            jax.ShapeDtypeStruct((pv, D), jnp.float32),
            jax.ShapeDtypeStruct((vv, D), jnp.float32),
            jax.ShapeDtypeStruct((2, D), jnp.float32),
        ),
    )(pitch_emb, velocity_emb, onset_W, onset_b.reshape(1, E), dur_W,
      dur_b.reshape(1, E), proj_W, proj_b.reshape(1, D))


_GDN = lax.GatherDimensionNumbers(offset_dims=(), collapsed_slice_dims=(0,),
                                  start_index_map=(0,))


def _dyn_gather(vec, idx):
    return lax.gather(vec, idx[:, None], _GDN, (1,),
                      mode=lax.GatherScatterMode.PROMISE_IN_BOUNDS)


def _splat(vec, lane):
    return _dyn_gather(vec, jnp.full((LANES,), lane, jnp.int32))


def _make_sc_kernel(nblk_total):
    assert nblk_total % NW == 0
    nblk = nblk_total // NW
    assert nblk % 2 == 0
    n_tokens = nblk_total * TBLK
    mesh = plsc.VectorSubcoreMesh(core_axis_name="c", subcore_axis_name="s",
                                  num_cores=NC, num_subcores=NS)

    @functools.partial(
        pl.kernel,
        out_type=jax.ShapeDtypeStruct((n_tokens, D), jnp.float32),
        mesh=mesh,
        scratch_types=[
            pltpu.VMEM((2, 2, TBLK), jnp.int32),
            pltpu.VMEM((2, 2, TBLK), jnp.float32),
            pltpu.VMEM((2, TBLK, D), jnp.float32),
            pltpu.VMEM((2, TBLK, D), jnp.float32),
            pltpu.VMEM((D,), jnp.float32),
            pltpu.VMEM((D,), jnp.float32),
            pltpu.VMEM_SHARED((128, D), jnp.float32),
            pltpu.VMEM_SHARED((128, D), jnp.float32),
            pltpu.SemaphoreType.DMA((2,)),
            pltpu.SemaphoreType.DMA((2,)),
            pltpu.SemaphoreType.DMA((2,)),
            pltpu.SemaphoreType.DMA((2,)),
            pltpu.SemaphoreType.DMA((2,)),
        ],
    )
    def sc_kernel(p2_hbm, v2_hbm, owdw_hbm, idx_hbm, sc_hbm, out_hbm,
                  idxb, scb, bufp, bufv, ow_v, dw_v, p2_sh, v2_sh,
                  sem_i, sem_s, sem_gp, sem_gv, sem_o):
        cid = lax.axis_index("c")
        sid = lax.axis_index("s")
        wid = sid * NC + cid

        @pl.when(sid == 0)
        def _stage_tables():
            pltpu.sync_copy(p2_hbm, bufp.at[0])
            pltpu.sync_copy(bufp.at[0], p2_sh)
            pltpu.sync_copy(v2_hbm, bufv.at[0])
            pltpu.sync_copy(bufv.at[0], v2_sh)

        pltpu.sync_copy(owdw_hbm.at[0], ow_v)
        pltpu.sync_copy(owdw_hbm.at[1], dw_v)
        plsc.subcore_barrier()
        ow_regs = [ow_v[pl.ds(LANES * j, LANES)] for j in range(NJ)]
        dw_regs = [dw_v[pl.ds(LANES * j, LANES)] for j in range(NJ)]
        blk0 = wid * nblk

        def i_copy(g, s):
            return pltpu.make_async_copy(idx_hbm.at[blk0 + g], idxb.at[s],
                                         sem_i.at[s])

        def s_copy(g, s):
            return pltpu.make_async_copy(sc_hbm.at[blk0 + g], scb.at[s],
                                         sem_s.at[s])

        def gp_copy(s):
            return pltpu.make_async_copy(p2_sh.at[idxb.at[s].at[0]],
                                         bufp.at[s], sem_gp.at[s])

        def gv_copy(s):
            return pltpu.make_async_copy(v2_sh.at[idxb.at[s].at[1]],
                                         bufv.at[s], sem_gv.at[s])

        def o_copy(g, s):
            return pltpu.make_async_copy(
                bufp.at[s],
                out_hbm.at[pl.ds((blk0 + g) * TBLK, TBLK)],
                sem_o.at[s])

        def compute(s):
            bp = bufp.at[s]
            bv = bufv.at[s]

            def group(t16, cc):
                tb = t16 * LANES
                ov = scb[s, 0, pl.ds(tb, LANES)]
                dv = scb[s, 1, pl.ds(tb, LANES)]
                for lane in range(LANES):
                    t = tb + lane
                    bo = _splat(ov, lane)
                    bd = _splat(dv, lane)
                    for j in range(NJ):
                        js = pl.ds(LANES * j, LANES)
                        bp[t, js] = (bp[t, js] + bv[t, js]
                                     + bo * ow_regs[j] + bd * dw_regs[j])
                return cc

            lax.fori_loop(0, TBLK // LANES, group, 0)

        i_copy(0, 0).start()
        s_copy(0, 0).start()
        i_copy(1, 1).start()
        s_copy(1, 1).start()
        i_copy(0, 0).wait()
        gp_copy(0).start()
        gv_copy(0).start()

        def block(gg, carry):
            for s in range(2):
                g = 2 * gg + s
                s1 = 1 - s

                @pl.when(g + 1 < nblk)
                def _next():
                    i_copy(g + 1, s1).wait()

                    @pl.when(g >= 1)
                    def _drain_out():
                        o_copy(g - 1, s1).wait()

                    gp_copy(s1).start()
                    gv_copy(s1).start()

                gp_copy(s).wait()
                gv_copy(s).wait()
                s_copy(g, s).wait()
                compute(s)

                @pl.when(g + 2 < nblk)
                def _fetch_next():
                    i_copy(g + 2, s).start()
                    s_copy(g + 2, s).start()

                o_copy(g, s).start()
            return carry

        lax.fori_loop(0, nblk // 2, block, 0)
        o_copy(nblk - 2, 0).wait()
        o_copy(nblk - 1, 1).wait()

    return sc_kernel


def kernel(x, pitch_emb, velocity_emb, onset_W, onset_b, dur_W, dur_b,
           proj_W, proj_b):
    b, l, _ = x.shape
    n = b * l
    nblk_total = n // TBLK
    p2, v2, aux = _prep(pitch_emb, velocity_emb, onset_W, onset_b, dur_W,
                        dur_b, proj_W, proj_b)
    xf = x.reshape(nblk_total, TBLK, 4)
    idx2 = jnp.stack([xf[:, :, 0], xf[:, :, 3]], axis=1).astype(jnp.int32)
    sc2 = jnp.stack([xf[:, :, 1], xf[:, :, 2]], axis=1)
    out = _make_sc_kernel(nblk_total)(p2, v2, aux, idx2, sc2)
    return out.reshape(b, l, D)


# speedup vs baseline: 15.8979x; 1.3589x over previous
import functools

import jax
import jax.numpy as jnp
from jax import lax
from jax.experimental import pallas as pl
from jax.experimental.pallas import tpu as pltpu
from jax.experimental.pallas import tpu_sc as plsc

NC, NS, LANES = 2, 16, 16
NW = NC * NS
D = 128
E = 64
TBLK = 128
NJ = D // LANES


def _prep_body(pitch_ref, vel_ref, ow_ref, ob_ref, dw_ref, db_ref, pw_ref,
               pb_ref, p2_ref, v2_ref, aux_ref):
    w1 = pw_ref[0:E, :]
    w2 = pw_ref[E:2 * E, :]
    w3 = pw_ref[2 * E:3 * E, :]
    w4 = pw_ref[3 * E:4 * E, :]
    const = (pb_ref[...]
             + jnp.dot(ob_ref[...], w2, preferred_element_type=jnp.float32)
             + jnp.dot(db_ref[...], w3, preferred_element_type=jnp.float32))
    p2_ref[...] = jnp.dot(pitch_ref[...], w1,
                          preferred_element_type=jnp.float32) + const
    v2_ref[...] = jnp.dot(vel_ref[...], w4,
                          preferred_element_type=jnp.float32)
    owv = jnp.dot(ow_ref[...], w2, preferred_element_type=jnp.float32)
    dwv = jnp.dot(dw_ref[...], w3, preferred_element_type=jnp.float32)
    aux_ref[...] = jnp.concatenate([owv, dwv], axis=0)


def _prep(pitch_emb, velocity_emb, onset_W, onset_b, dur_W, dur_b, proj_W,
          proj_b):
    pv = pitch_emb.shape[0]
    vv = velocity_emb.shape[0]
    return pl.pallas_call(
        _prep_body,
        out_shape=(
            jax.ShapeDtypeStruct((pv, D), jnp.float32),
            jax.ShapeDtypeStruct((vv, D), jnp.float32),
            jax.ShapeDtypeStruct((2, D), jnp.float32),
        ),
    )(pitch_emb, velocity_emb, onset_W, onset_b.reshape(1, E), dur_W,
      dur_b.reshape(1, E), proj_W, proj_b.reshape(1, D))


_GDN = lax.GatherDimensionNumbers(offset_dims=(), collapsed_slice_dims=(0,),
                                  start_index_map=(0,))


def _dyn_gather(vec, idx):
    return lax.gather(vec, idx[:, None], _GDN, (1,),
                      mode=lax.GatherScatterMode.PROMISE_IN_BOUNDS)


def _splat(vec, lane):
    return _dyn_gather(vec, jnp.full((LANES,), lane, jnp.int32))


SLOTS = 3


def _make_sc_kernel(nblk_total):
    assert nblk_total % NW == 0
    nblk = nblk_total // NW
    n_tokens = nblk_total * TBLK
    nloop = (nblk - 2) // SLOTS
    assert nloop * SLOTS + 2 == nblk
    mesh = plsc.VectorSubcoreMesh(core_axis_name="c", subcore_axis_name="s",
                                  num_cores=NC, num_subcores=NS)

    @functools.partial(
        pl.kernel,
        out_type=jax.ShapeDtypeStruct((n_tokens, D), jnp.float32),
        mesh=mesh,
        scratch_types=[
            pltpu.VMEM((SLOTS, 2, TBLK), jnp.int32),
            pltpu.VMEM((SLOTS, 2, TBLK), jnp.float32),
            pltpu.VMEM((SLOTS, TBLK, D), jnp.float32),
            pltpu.VMEM((D,), jnp.float32),
            pltpu.VMEM((D,), jnp.float32),
            pltpu.VMEM_SHARED((128, D), jnp.float32),
            pltpu.VMEM_SHARED((128, D), jnp.float32),
            pltpu.SemaphoreType.DMA((SLOTS,)),
            pltpu.SemaphoreType.DMA((SLOTS,)),
            pltpu.SemaphoreType.DMA((SLOTS,)),
            pltpu.SemaphoreType.DMA((SLOTS,)),
            pltpu.SemaphoreType.DMA((SLOTS,)),
        ],
    )
    def sc_kernel(p2_hbm, v2_hbm, owdw_hbm, idx_hbm, sc_hbm, out_hbm,
                  idxb, scb, bufp, ow_v, dw_v, p2_sh, v2_sh,
                  sem_i, sem_s, sem_gp, sem_gv, sem_o):
        cid = lax.axis_index("c")
        sid = lax.axis_index("s")
        wid = sid * NC + cid

        @pl.when(sid == 0)
        def _stage_tables():
            pltpu.sync_copy(p2_hbm, bufp.at[0])
            pltpu.sync_copy(bufp.at[0], p2_sh)
            pltpu.sync_copy(v2_hbm, bufp.at[1])
            pltpu.sync_copy(bufp.at[1], v2_sh)

        pltpu.sync_copy(owdw_hbm.at[0], ow_v)
        pltpu.sync_copy(owdw_hbm.at[1], dw_v)
        plsc.subcore_barrier()
        ow_regs = [ow_v[pl.ds(LANES * j, LANES)] for j in range(NJ)]
        dw_regs = [dw_v[pl.ds(LANES * j, LANES)] for j in range(NJ)]
        blk0 = wid * nblk

        def i_copy(g, s):
            return pltpu.make_async_copy(idx_hbm.at[blk0 + g], idxb.at[s],
                                         sem_i.at[s])

        def s_copy(g, s):
            return pltpu.make_async_copy(sc_hbm.at[blk0 + g], scb.at[s],
                                         sem_s.at[s])

        def gp_copy(s):
            return pltpu.make_async_copy(p2_sh.at[idxb.at[s].at[0]],
                                         bufp.at[s], sem_gp.at[s])

        def gv_fire(s):
            pltpu.async_copy(v2_sh.at[idxb.at[s].at[1]], bufp.at[s],
                             sem_gv.at[s], add=True)

        def gv_wait(s):
            pltpu.make_async_copy(v2_sh.at[idxb.at[s].at[1]], bufp.at[s],
                                  sem_gv.at[s]).wait()

        def o_copy(g, s):
            return pltpu.make_async_copy(
                bufp.at[s],
                out_hbm.at[pl.ds((blk0 + g) * TBLK, TBLK)],
                sem_o.at[s])

        def compute(s):
            bp = bufp.at[s]

            def group(t16, cc):
                tb = t16 * LANES
                ov = scb[s, 0, pl.ds(tb, LANES)]
                dv = scb[s, 1, pl.ds(tb, LANES)]
                for lane in range(LANES):
                    t = tb + lane
                    bo = _splat(ov, lane)
                    bd = _splat(dv, lane)
                    for j in range(NJ):
                        js = pl.ds(LANES * j, LANES)
                        bp[t, js] = (bp[t, js]
                                     + bo * ow_regs[j] + bd * dw_regs[j])
                return cc

            lax.fori_loop(0, TBLK // LANES, group, 0)

        def emit_iter(g, sl):
            sl1 = (sl + 1) % SLOTS
            sl2 = (sl + 2) % SLOTS

            @pl.when(g + 2 < nblk)
            def _fire_gp():
                i_copy(g + 2, sl2).wait()

                @pl.when(g >= 1)
                def _drain_out():
                    o_copy(g - 1, sl2).wait()

                gp_copy(sl2).start()

            @pl.when(g + 1 < nblk)
            def _fire_gv():
                gp_copy(sl1).wait()
                gv_fire(sl1)

            gv_wait(sl)
            s_copy(g, sl).wait()
            compute(sl)

            @pl.when(g + 3 < nblk)
            def _fetch_next():
                i_copy(g + 3, sl).start()
                s_copy(g + 3, sl).start()

            o_copy(g, sl).start()

        for t in range(SLOTS):
            i_copy(t, t).start()
            s_copy(t, t).start()
        i_copy(0, 0).wait()
        gp_copy(0).start()
        i_copy(1, 1).wait()
        gp_copy(1).start()
        gp_copy(0).wait()
        gv_fire(0)

        def block(gg, carry):
            for k in range(SLOTS):
                emit_iter(SLOTS * gg + k, k)
            return carry

        lax.fori_loop(0, nloop, block, 0)
        emit_iter(jnp.int32(nblk - 2), (nblk - 2) % SLOTS)
        emit_iter(jnp.int32(nblk - 1), (nblk - 1) % SLOTS)
        for g in (nblk - 3, nblk - 2, nblk - 1):
            o_copy(g, g % SLOTS).wait()

    return sc_kernel


def kernel(x, pitch_emb, velocity_emb, onset_W, onset_b, dur_W, dur_b,
           proj_W, proj_b):
    b, l, _ = x.shape
    n = b * l
    nblk_total = n // TBLK
    p2, v2, aux = _prep(pitch_emb, velocity_emb, onset_W, onset_b, dur_W,
                        dur_b, proj_W, proj_b)
    xf = x.reshape(nblk_total, TBLK, 4)
    idx2 = jnp.stack([xf[:, :, 0], xf[:, :, 3]], axis=1).astype(jnp.int32)
    sc2 = jnp.stack([xf[:, :, 1], xf[:, :, 2]], axis=1)
    out = _make_sc_kernel(nblk_total)(p2, v2, aux, idx2, sc2)
    return out.reshape(b, l, D)
